# baseline (speedup 1.0000x reference)
import functools

import jax
import jax.numpy as jnp
from jax import lax
from jax.experimental import pallas as pl
from jax.experimental.pallas import tpu as pltpu
from jax.experimental.pallas import tpu_sc as plsc

GAMMA = 0.1
EPS = 0.1

_CHUNK = 128
_NC = 2
_NS = 16
_NW = _NC * _NS
_BN = 1000


def _pad_chunks(nchunk):
    cpw = -(-nchunk // _NW)
    return -(-cpw // 8) * 8


def _pad_nodes(n_nodes):
    per_tile = -(-n_nodes // _NS)
    per_tile = -(-per_tile // 128) * 128
    return per_tile * _NS, per_tile


_IB = 16


def _scatter_add_sc(y, row_r, col_r, zeros, nchunk_real, n_pad, rpt):
    nchunk_pad = row_r.shape[0]
    d = y.shape[1]
    cpw = nchunk_pad // _NW
    nstages = cpw // _IB
    assert nstages * _IB == cpw

    mesh = plsc.VectorSubcoreMesh(core_axis_name="c", subcore_axis_name="s")

    @functools.partial(
        pl.kernel,
        mesh=mesh,
        out_type=jax.ShapeDtypeStruct((2 * n_pad, d), jnp.float32),
        scratch_types=[
            pltpu.VMEM((2, _IB, _CHUNK), jnp.int32),
            pltpu.VMEM((2, _IB, _CHUNK), jnp.int32),
            pltpu.VMEM((2, _CHUNK, d), jnp.float32),
            pltpu.VMEM_SHARED((n_pad, d), jnp.float32),
            pltpu.SemaphoreType.DMA,
            pltpu.SemaphoreType.DMA,
            pltpu.SemaphoreType.DMA,
            pltpu.SemaphoreType.DMA,
            pltpu.SemaphoreType.DMA,
        ],
    )
    def scatter_kernel(y_hbm, row_hbm, col_hbm, z_hbm, out_hbm,
                       ridx, cidx, bufs, acc, sem0, sem1, sem_i,
                       ssem0, ssem1):
        cid = lax.axis_index("c")
        sid = lax.axis_index("s")
        wid = sid * _NC + cid
        sems = (sem0, sem1)
        ssems = (ssem0, ssem1)

        base = pl.multiple_of(sid * rpt, 8)
        pltpu.sync_copy(z_hbm, acc.at[pl.ds(base, rpt)])
        plsc.subcore_barrier()

        lo = pl.multiple_of(wid * cpw, 8)
        cnt = lax.max(0, lax.min(nchunk_real - lo, cpw))

        def _istart(t, tb):
            off = pl.multiple_of(lo + t * _IB, 8)
            pltpu.async_copy(row_hbm.at[pl.ds(off, _IB)], ridx.at[tb], sem_i)
            pltpu.async_copy(col_hbm.at[pl.ds(off, _IB)], cidx.at[tb], sem_i)

        def _iwait(t, tb):
            off = pl.multiple_of(lo + t * _IB, 8)
            pltpu.make_async_copy(row_hbm.at[pl.ds(off, _IB)], ridx.at[tb],
                                  sem_i).wait()
            pltpu.make_async_copy(col_hbm.at[pl.ds(off, _IB)], cidx.at[tb],
                                  sem_i).wait()

        def _gstart(tb, i, b):
            pltpu.async_copy(y_hbm.at[ridx.at[tb, i]], bufs.at[b], sems[b])

        def _gwait(tb, i, b):
            pltpu.make_async_copy(y_hbm.at[ridx.at[tb, i]], bufs.at[b],
                                  sems[b]).wait()

        def _sstart(tb, i, b):
            pltpu.async_copy(bufs.at[b], acc.at[cidx.at[tb, i]], ssems[b],
                             add=True)

        def _swait(tb, i, b):
            pltpu.make_async_copy(bufs.at[b], acc.at[cidx.at[tb, i]],
                                  ssems[b]).wait()

        _istart(0, 0)
        for t in range(nstages):
            tb = t % 2
            _iwait(t, tb)
            if t + 1 < nstages:
                _istart(t + 1, 1 - tb)
            cnt_t = lax.max(0, lax.min(cnt - t * _IB, _IB))

            @pl.when(cnt_t > 0)
            def _():
                _gstart(tb, 0, 0)

            def _step(i2, _):
                for b in range(2):
                    i = i2 * 2 + b

                    @pl.when(i < cnt_t)
                    def _():
                        _gwait(tb, i, b)

                        @pl.when(i >= 1)
                        def _():
                            _swait(tb, i, 1 - b)

                        @pl.when(i + 1 < cnt_t)
                        def _():
                            _gstart(tb, i + 1, 1 - b)

                        _sstart(tb, i, b)
                return 0
            lax.fori_loop(0, _IB // 2, _step, 0)

            @pl.when((cnt_t > 0) & (cnt_t % 2 == 1))
            def _():
                _swait(tb, 0, 0)

            @pl.when((cnt_t > 0) & (cnt_t % 2 == 0))
            def _():
                _swait(tb, 0, 1)
        plsc.subcore_barrier()

        out_base = pl.multiple_of(cid * n_pad + base, 8)
        pltpu.sync_copy(acc.at[pl.ds(base, rpt)],
                        out_hbm.at[pl.ds(out_base, rpt)])

    return scatter_kernel(y, row_r, col_r, zeros)


def _deg_sc(row_r, ones16, zeros16, nchunk_real, n_pad, rpt):
    nchunk_pad = row_r.shape[0]
    cpw = nchunk_pad // _NW
    nstages = cpw // _IB
    assert nstages * _IB == cpw

    mesh = plsc.VectorSubcoreMesh(core_axis_name="c", subcore_axis_name="s")

    @functools.partial(
        pl.kernel,
        mesh=mesh,
        out_type=jax.ShapeDtypeStruct((2 * n_pad, 128), jnp.float32),
        scratch_types=[
            pltpu.VMEM((2, _IB, _CHUNK), jnp.int32),
            pltpu.VMEM((_CHUNK, 128), jnp.float32),
            pltpu.VMEM_SHARED((n_pad, 128), jnp.float32),
            pltpu.SemaphoreType.DMA,
        ],
    )
    def deg_kernel(row_hbm, ones_hbm, z_hbm, out_hbm, ridx, onesb, acc, sem_i):
        cid = lax.axis_index("c")
        sid = lax.axis_index("s")
        wid = sid * _NC + cid

        base = pl.multiple_of(sid * rpt, 8)
        pltpu.sync_copy(z_hbm, acc.at[pl.ds(base, rpt)])
        pltpu.sync_copy(ones_hbm, onesb)
        plsc.subcore_barrier()

        lo = pl.multiple_of(wid * cpw, 8)
        cnt = lax.max(0, lax.min(nchunk_real - lo, cpw))

        def _istart(t, tb):
            off = pl.multiple_of(lo + t * _IB, 8)
            pltpu.async_copy(row_hbm.at[pl.ds(off, _IB)], ridx.at[tb], sem_i)

        def _iwait(t, tb):
            off = pl.multiple_of(lo + t * _IB, 8)
            pltpu.make_async_copy(row_hbm.at[pl.ds(off, _IB)], ridx.at[tb],
                                  sem_i).wait()

        _istart(0, 0)
        for t in range(nstages):
            tb = t % 2
            _iwait(t, tb)
            if t + 1 < nstages:
                _istart(t + 1, 1 - tb)
            cnt_t = lax.max(0, lax.min(cnt - t * _IB, _IB))

            def _step(i, _):
                @pl.when(i < cnt_t)
                def _():
                    pltpu.sync_copy(onesb, acc.at[ridx.at[tb, i]], add=True)
                return 0
            lax.fori_loop(0, _IB, _step, 0)
        plsc.subcore_barrier()

        out_base = pl.multiple_of(cid * n_pad + base, 8)
        pltpu.sync_copy(acc.at[pl.ds(base, rpt)],
                        out_hbm.at[pl.ds(out_base, rpt)])

    return deg_kernel(row_r, ones16, zeros16)


def _embed_tc(x, w_t, deg):
    n, d = x.shape

    def body(x_ref, w_ref, deg_ref, h_ref, dis_ref):
        h_ref[...] = jnp.dot(x_ref[...], w_ref[...],
                             preferred_element_type=jnp.float32)
        dg = deg_ref[...]
        dis_ref[...] = jnp.where(dg > 0, lax.rsqrt(dg), 0.0)

    return pl.pallas_call(
        body,
        grid=(n // _BN,),
        in_specs=[
            pl.BlockSpec((_BN, d), lambda i: (i, 0)),
            pl.BlockSpec((d, d), lambda i: (0, 0)),
            pl.BlockSpec((_BN, 1), lambda i: (i, 0)),
        ],
        out_specs=[
            pl.BlockSpec((_BN, d), lambda i: (i, 0)),
            pl.BlockSpec((_BN, 1), lambda i: (i, 0)),
        ],
        out_shape=[
            jax.ShapeDtypeStruct((n, d), jnp.float32),
            jax.ShapeDtypeStruct((n, 1), jnp.float32),
        ],
    )(x, w_t, deg)


def _proj_tc(h, cmat, dis):
    n, d = h.shape

    def body(h_ref, c_ref, dis_ref, hw_ref, y_ref):
        z = jnp.dot(h_ref[...], c_ref[...],
                    preferred_element_type=jnp.float32)
        hw_ref[...] = z[:, :d]
        y_ref[...] = dis_ref[...] * z[:, d:]

    return pl.pallas_call(
        body,
        grid=(n // _BN,),
        in_specs=[
            pl.BlockSpec((_BN, d), lambda i: (i, 0)),
            pl.BlockSpec((d, 2 * d), lambda i: (0, 0)),
            pl.BlockSpec((_BN, 1), lambda i: (i, 0)),
        ],
        out_specs=[
            pl.BlockSpec((_BN, d), lambda i: (i, 0)),
            pl.BlockSpec((_BN, d), lambda i: (i, 0)),
        ],
        out_shape=[
            jax.ShapeDtypeStruct((n, d), jnp.float32),
            jax.ShapeDtypeStruct((n, d), jnp.float32),
        ],
    )(h, cmat, dis)


def _update_tc(h, hw, acc, dis, bias):
    n, d = h.shape

    def body(h_ref, hw_ref, acc_ref, dis_ref, b_ref, o_ref):
        agg = dis_ref[...] * (acc_ref[0] + acc_ref[1])
        o_ref[...] = h_ref[...] + EPS * jnp.tanh(hw_ref[...] + agg + b_ref[...])

    return pl.pallas_call(
        body,
        grid=(n // _BN,),
        in_specs=[
            pl.BlockSpec((_BN, d), lambda i: (i, 0)),
            pl.BlockSpec((_BN, d), lambda i: (i, 0)),
            pl.BlockSpec((2, _BN, d), lambda i: (0, i, 0)),
            pl.BlockSpec((_BN, 1), lambda i: (i, 0)),
            pl.BlockSpec((1, d), lambda i: (0, 0)),
        ],
        out_specs=pl.BlockSpec((_BN, d), lambda i: (i, 0)),
        out_shape=jax.ShapeDtypeStruct((n, d), jnp.float32),
    )(h, hw, acc, dis, bias)


def kernel(x, edge_index, emb_w, Weights, biases, lin_ws):
    n, d = x.shape
    e = edge_index.shape[1]
    nlayers = Weights.shape[0]

    nchunk = e // _CHUNK
    cpw = _pad_chunks(nchunk)
    nchunk_pad = cpw * _NW
    n_pad, rpt = _pad_nodes(n)

    row_r = edge_index[0].reshape(nchunk, _CHUNK)
    col_r = edge_index[1].reshape(nchunk, _CHUNK)
    pad = ((0, nchunk_pad - nchunk), (0, 0))
    row_r = jnp.pad(row_r, pad)
    col_r = jnp.pad(col_r, pad)

    eye = jnp.eye(d, dtype=jnp.float32)
    wts = jnp.transpose(Weights, (0, 2, 1)) - Weights - GAMMA * eye
    cmats = jnp.concatenate([wts, jnp.transpose(lin_ws, (0, 2, 1))], axis=2)

    zeros = jnp.zeros((rpt, d), jnp.float32)
    ones_c = jnp.ones((_CHUNK, d), jnp.float32)
    degbuf = _deg_sc(row_r, ones_c, zeros, nchunk, n_pad, rpt)
    deg = (degbuf[:n, 0] + degbuf[n_pad:n_pad + n, 0]).reshape(n, 1)

    h, dis = _embed_tc(x, emb_w.T, deg)
    for l in range(nlayers):
        hw, y = _proj_tc(h, cmats[l], dis)
        accp = _scatter_add_sc(y, row_r, col_r, zeros, nchunk, n_pad, rpt)
        acc = jnp.stack([accp[:n], accp[n_pad:n_pad + n]])
        h = _update_tc(h, hw, acc, dis, biases[l].reshape(1, d))
    return h


# speedup vs baseline: 1.0549x; 1.0549x over previous
import functools

import jax
import jax.numpy as jnp
from jax import lax
from jax.experimental import pallas as pl
from jax.experimental.pallas import tpu as pltpu
from jax.experimental.pallas import tpu_sc as plsc

GAMMA = 0.1
EPS = 0.1

_CHUNK = 128
_NC = 2
_NS = 16
_NW = _NC * _NS
_BN = 1000


def _pad_chunks(nchunk):
    cpw = -(-nchunk // _NW)
    return -(-cpw // 8) * 8


def _pad_nodes(n_nodes):
    per_tile = -(-n_nodes // _NS)
    per_tile = -(-per_tile // 128) * 128
    return per_tile * _NS, per_tile


_IB = 16


def _scatter_add_sc(y, row_r, col_r, zeros, nchunk_real, n_pad, rpt):
    nchunk_pad = row_r.shape[0]
    d = y.shape[1]
    cpw = nchunk_pad // _NW
    nstages = cpw // _IB
    assert nstages * _IB == cpw

    mesh = plsc.VectorSubcoreMesh(core_axis_name="c", subcore_axis_name="s")

    @functools.partial(
        pl.kernel,
        mesh=mesh,
        out_type=jax.ShapeDtypeStruct((2 * n_pad, d), jnp.float32),
        scratch_types=[
            pltpu.VMEM((2, _IB, _CHUNK), jnp.int32),
            pltpu.VMEM((2, _IB, _CHUNK), jnp.int32),
            pltpu.VMEM((2, _CHUNK, d), jnp.float32),
            pltpu.VMEM_SHARED((n_pad, d), jnp.float32),
            pltpu.SemaphoreType.DMA,
            pltpu.SemaphoreType.DMA,
            pltpu.SemaphoreType.DMA,
            pltpu.SemaphoreType.DMA,
            pltpu.SemaphoreType.DMA,
        ],
    )
    def scatter_kernel(y_hbm, row_hbm, col_hbm, z_hbm, out_hbm,
                       ridx, cidx, bufs, acc, sem0, sem1, sem_i,
                       ssem0, ssem1):
        cid = lax.axis_index("c")
        sid = lax.axis_index("s")
        wid = sid * _NC + cid
        sems = (sem0, sem1)
        ssems = (ssem0, ssem1)

        base = pl.multiple_of(sid * rpt, 8)
        pltpu.sync_copy(z_hbm, acc.at[pl.ds(base, rpt)])
        plsc.subcore_barrier()

        lo = pl.multiple_of(wid * cpw, 8)
        cnt = lax.max(0, lax.min(nchunk_real - lo, cpw))

        def _istart(t, tb):
            off = pl.multiple_of(lo + t * _IB, 8)
            pltpu.async_copy(row_hbm.at[pl.ds(off, _IB)], ridx.at[tb], sem_i)
            pltpu.async_copy(col_hbm.at[pl.ds(off, _IB)], cidx.at[tb], sem_i)

        def _iwait(t, tb):
            off = pl.multiple_of(lo + t * _IB, 8)
            pltpu.make_async_copy(row_hbm.at[pl.ds(off, _IB)], ridx.at[tb],
                                  sem_i).wait()
            pltpu.make_async_copy(col_hbm.at[pl.ds(off, _IB)], cidx.at[tb],
                                  sem_i).wait()

        def _gstart(tb, i, b):
            pltpu.async_copy(y_hbm.at[ridx.at[tb, i]], bufs.at[b], sems[b])

        def _gwait(tb, i, b):
            pltpu.make_async_copy(y_hbm.at[ridx.at[tb, i]], bufs.at[b],
                                  sems[b]).wait()

        def _sstart(tb, i, b):
            pltpu.async_copy(bufs.at[b], acc.at[cidx.at[tb, i]], ssems[b],
                             add=True)

        def _swait(tb, i, b):
            pltpu.make_async_copy(bufs.at[b], acc.at[cidx.at[tb, i]],
                                  ssems[b]).wait()

        _istart(0, 0)
        for t in range(nstages):
            tb = t % 2
            _iwait(t, tb)
            if t + 1 < nstages:
                _istart(t + 1, 1 - tb)
            cnt_t = lax.max(0, lax.min(cnt - t * _IB, _IB))

            @pl.when(cnt_t > 0)
            def _():
                _gstart(tb, 0, 0)

            def _step(i2, _):
                for b in range(2):
                    i = i2 * 2 + b

                    @pl.when(i < cnt_t)
                    def _():
                        _gwait(tb, i, b)

                        @pl.when(i >= 1)
                        def _():
                            _swait(tb, i, 1 - b)

                        @pl.when(i + 1 < cnt_t)
                        def _():
                            _gstart(tb, i + 1, 1 - b)

                        _sstart(tb, i, b)
                return 0
            lax.fori_loop(0, _IB // 2, _step, 0)

            @pl.when((cnt_t > 0) & (cnt_t % 2 == 1))
            def _():
                _swait(tb, 0, 0)

            @pl.when((cnt_t > 0) & (cnt_t % 2 == 0))
            def _():
                _swait(tb, 0, 1)
        plsc.subcore_barrier()

        out_base = pl.multiple_of(cid * n_pad + base, 8)
        pltpu.sync_copy(acc.at[pl.ds(base, rpt)],
                        out_hbm.at[pl.ds(out_base, rpt)])

    return scatter_kernel(y, row_r, col_r, zeros)


def _deg_sc(row_r, ones16, zeros16, nchunk_real, n_pad, rpt):
    nchunk_pad = row_r.shape[0]
    cpw = nchunk_pad // _NW
    nstages = cpw // _IB
    assert nstages * _IB == cpw

    mesh = plsc.VectorSubcoreMesh(core_axis_name="c", subcore_axis_name="s")

    @functools.partial(
        pl.kernel,
        mesh=mesh,
        out_type=jax.ShapeDtypeStruct((2 * n_pad, 128), jnp.float32),
        scratch_types=[
            pltpu.VMEM((2, _IB, _CHUNK), jnp.int32),
            pltpu.VMEM((_CHUNK, 128), jnp.float32),
            pltpu.VMEM_SHARED((n_pad, 128), jnp.float32),
            pltpu.SemaphoreType.DMA,
        ],
    )
    def deg_kernel(row_hbm, ones_hbm, z_hbm, out_hbm, ridx, onesb, acc, sem_i):
        cid = lax.axis_index("c")
        sid = lax.axis_index("s")
        wid = sid * _NC + cid

        base = pl.multiple_of(sid * rpt, 8)
        pltpu.sync_copy(z_hbm, acc.at[pl.ds(base, rpt)])
        pltpu.sync_copy(ones_hbm, onesb)
        plsc.subcore_barrier()

        lo = pl.multiple_of(wid * cpw, 8)
        cnt = lax.max(0, lax.min(nchunk_real - lo, cpw))

        def _istart(t, tb):
            off = pl.multiple_of(lo + t * _IB, 8)
            pltpu.async_copy(row_hbm.at[pl.ds(off, _IB)], ridx.at[tb], sem_i)

        def _iwait(t, tb):
            off = pl.multiple_of(lo + t * _IB, 8)
            pltpu.make_async_copy(row_hbm.at[pl.ds(off, _IB)], ridx.at[tb],
                                  sem_i).wait()

        _istart(0, 0)
        for t in range(nstages):
            tb = t % 2
            _iwait(t, tb)
            if t + 1 < nstages:
                _istart(t + 1, 1 - tb)
            cnt_t = lax.max(0, lax.min(cnt - t * _IB, _IB))

            def _step(i, _):
                @pl.when(i < cnt_t)
                def _():
                    pltpu.sync_copy(onesb, acc.at[ridx.at[tb, i]], add=True)
                return 0
            lax.fori_loop(0, _IB, _step, 0)
        plsc.subcore_barrier()

        out_base = pl.multiple_of(cid * n_pad + base, 8)
        pltpu.sync_copy(acc.at[pl.ds(base, rpt)],
                        out_hbm.at[pl.ds(out_base, rpt)])

    return deg_kernel(row_r, ones16, zeros16)


def _embed_proj_tc(x, w_t, cmat, deg):
    n, d = x.shape

    def body(x_ref, w_ref, c_ref, deg_ref, h_ref, dis_ref, hw_ref, y_ref):
        h = jnp.dot(x_ref[...], w_ref[...],
                    preferred_element_type=jnp.float32)
        h_ref[...] = h
        dg = deg_ref[...]
        dis = jnp.where(dg > 0, lax.rsqrt(dg), 0.0)
        dis_ref[...] = dis
        z = jnp.dot(h, c_ref[...], preferred_element_type=jnp.float32)
        hw_ref[...] = z[:, :d]
        y_ref[...] = dis * z[:, d:]

    return pl.pallas_call(
        body,
        grid=(n // _BN,),
        in_specs=[
            pl.BlockSpec((_BN, d), lambda i: (i, 0)),
            pl.BlockSpec((d, d), lambda i: (0, 0)),
            pl.BlockSpec((d, 2 * d), lambda i: (0, 0)),
            pl.BlockSpec((_BN, 1), lambda i: (i, 0)),
        ],
        out_specs=[
            pl.BlockSpec((_BN, d), lambda i: (i, 0)),
            pl.BlockSpec((_BN, 1), lambda i: (i, 0)),
            pl.BlockSpec((_BN, d), lambda i: (i, 0)),
            pl.BlockSpec((_BN, d), lambda i: (i, 0)),
        ],
        out_shape=[
            jax.ShapeDtypeStruct((n, d), jnp.float32),
            jax.ShapeDtypeStruct((n, 1), jnp.float32),
            jax.ShapeDtypeStruct((n, d), jnp.float32),
            jax.ShapeDtypeStruct((n, d), jnp.float32),
        ],
    )(x, w_t, cmat, deg)


def _update_proj_tc(h, hw, acc, dis, bias, cmat):
    n, d = h.shape

    def body(h_ref, hw_ref, acc_ref, dis_ref, b_ref, c_ref,
             o_ref, hw2_ref, y_ref):
        dis = dis_ref[...]
        agg = dis * (acc_ref[0] + acc_ref[1])
        hn = h_ref[...] + EPS * jnp.tanh(hw_ref[...] + agg + b_ref[...])
        o_ref[...] = hn
        z = jnp.dot(hn, c_ref[...], preferred_element_type=jnp.float32)
        hw2_ref[...] = z[:, :d]
        y_ref[...] = dis * z[:, d:]

    return pl.pallas_call(
        body,
        grid=(n // _BN,),
        in_specs=[
            pl.BlockSpec((_BN, d), lambda i: (i, 0)),
            pl.BlockSpec((_BN, d), lambda i: (i, 0)),
            pl.BlockSpec((2, _BN, d), lambda i: (0, i, 0)),
            pl.BlockSpec((_BN, 1), lambda i: (i, 0)),
            pl.BlockSpec((1, d), lambda i: (0, 0)),
            pl.BlockSpec((d, 2 * d), lambda i: (0, 0)),
        ],
        out_specs=[
            pl.BlockSpec((_BN, d), lambda i: (i, 0)),
            pl.BlockSpec((_BN, d), lambda i: (i, 0)),
            pl.BlockSpec((_BN, d), lambda i: (i, 0)),
        ],
        out_shape=[
            jax.ShapeDtypeStruct((n, d), jnp.float32),
            jax.ShapeDtypeStruct((n, d), jnp.float32),
            jax.ShapeDtypeStruct((n, d), jnp.float32),
        ],
    )(h, hw, acc, dis, bias, cmat)


def _proj_tc(h, cmat, dis):
    n, d = h.shape

    def body(h_ref, c_ref, dis_ref, hw_ref, y_ref):
        z = jnp.dot(h_ref[...], c_ref[...],
                    preferred_element_type=jnp.float32)
        hw_ref[...] = z[:, :d]
        y_ref[...] = dis_ref[...] * z[:, d:]

    return pl.pallas_call(
        body,
        grid=(n // _BN,),
        in_specs=[
            pl.BlockSpec((_BN, d), lambda i: (i, 0)),
            pl.BlockSpec((d, 2 * d), lambda i: (0, 0)),
            pl.BlockSpec((_BN, 1), lambda i: (i, 0)),
        ],
        out_specs=[
            pl.BlockSpec((_BN, d), lambda i: (i, 0)),
            pl.BlockSpec((_BN, d), lambda i: (i, 0)),
        ],
        out_shape=[
            jax.ShapeDtypeStruct((n, d), jnp.float32),
            jax.ShapeDtypeStruct((n, d), jnp.float32),
        ],
    )(h, cmat, dis)


def _update_tc(h, hw, acc, dis, bias):
    n, d = h.shape

    def body(h_ref, hw_ref, acc_ref, dis_ref, b_ref, o_ref):
        agg = dis_ref[...] * (acc_ref[0] + acc_ref[1])
        o_ref[...] = h_ref[...] + EPS * jnp.tanh(hw_ref[...] + agg + b_ref[...])

    return pl.pallas_call(
        body,
        grid=(n // _BN,),
        in_specs=[
            pl.BlockSpec((_BN, d), lambda i: (i, 0)),
            pl.BlockSpec((_BN, d), lambda i: (i, 0)),
            pl.BlockSpec((2, _BN, d), lambda i: (0, i, 0)),
            pl.BlockSpec((_BN, 1), lambda i: (i, 0)),
            pl.BlockSpec((1, d), lambda i: (0, 0)),
        ],
        out_specs=pl.BlockSpec((_BN, d), lambda i: (i, 0)),
        out_shape=jax.ShapeDtypeStruct((n, d), jnp.float32),
    )(h, hw, acc, dis, bias)


def kernel(x, edge_index, emb_w, Weights, biases, lin_ws):
    n, d = x.shape
    e = edge_index.shape[1]
    nlayers = Weights.shape[0]

    nchunk = e // _CHUNK
    cpw = _pad_chunks(nchunk)
    nchunk_pad = cpw * _NW
    n_pad, rpt = _pad_nodes(n)

    row_r = edge_index[0].reshape(nchunk, _CHUNK)
    col_r = edge_index[1].reshape(nchunk, _CHUNK)
    pad = ((0, nchunk_pad - nchunk), (0, 0))
    row_r = jnp.pad(row_r, pad)
    col_r = jnp.pad(col_r, pad)

    eye = jnp.eye(d, dtype=jnp.float32)
    wts = jnp.transpose(Weights, (0, 2, 1)) - Weights - GAMMA * eye
    cmats = jnp.concatenate([wts, jnp.transpose(lin_ws, (0, 2, 1))], axis=2)

    zeros = jnp.zeros((rpt, d), jnp.float32)
    ones_c = jnp.ones((_CHUNK, d), jnp.float32)
    degbuf = _deg_sc(row_r, ones_c, zeros, nchunk, n_pad, rpt)
    deg = (degbuf[:n, 0] + degbuf[n_pad:n_pad + n, 0]).reshape(n, 1)

    h, dis, hw, y = _embed_proj_tc(x, emb_w.T, cmats[0], deg)
    for l in range(nlayers):
        accp = _scatter_add_sc(y, row_r, col_r, zeros, nchunk, n_pad, rpt)
        acc = jnp.stack([accp[:n], accp[n_pad:n_pad + n]])
        if l + 1 < nlayers:
            h, hw, y = _update_proj_tc(h, hw, acc, dis,
                                       biases[l].reshape(1, d), cmats[l + 1])
        else:
            h = _update_tc(h, hw, acc, dis, biases[l].reshape(1, d))
    return h


# speedup vs baseline: 1.0597x; 1.0046x over previous
import functools

import jax
import jax.numpy as jnp
from jax import lax
from jax.experimental import pallas as pl
from jax.experimental.pallas import tpu as pltpu
from jax.experimental.pallas import tpu_sc as plsc

GAMMA = 0.1
EPS = 0.1

_CHUNK = 128
_NC = 2
_NS = 16
_NW = _NC * _NS
_BN = 1000


def _pad_chunks(nchunk):
    cpw = -(-nchunk // _NW)
    return -(-cpw // 8) * 8


def _pad_nodes(n_nodes):
    per_tile = -(-n_nodes // _NS)
    per_tile = -(-per_tile // 128) * 128
    return per_tile * _NS, per_tile


_IB = 16


def _scatter_add_sc(y, row_r, col_r, zeros, nchunk_real, n_pad, rpt):
    nchunk_pad = row_r.shape[0]
    d = y.shape[1]
    cpw = nchunk_pad // _NW
    nstages = cpw // _IB
    assert nstages * _IB == cpw

    mesh = plsc.VectorSubcoreMesh(core_axis_name="c", subcore_axis_name="s")

    @functools.partial(
        pl.kernel,
        mesh=mesh,
        out_type=jax.ShapeDtypeStruct((2 * n_pad, d), jnp.float32),
        scratch_types=[
            pltpu.VMEM((2, _IB, _CHUNK), jnp.int32),
            pltpu.VMEM((2, _IB, _CHUNK), jnp.int32),
            pltpu.VMEM((2, _CHUNK, d), jnp.float32),
            pltpu.VMEM_SHARED((n_pad, d), jnp.float32),
            pltpu.SemaphoreType.DMA,
            pltpu.SemaphoreType.DMA,
            pltpu.SemaphoreType.DMA,
            pltpu.SemaphoreType.DMA,
            pltpu.SemaphoreType.DMA,
        ],
    )
    def scatter_kernel(y_hbm, row_hbm, col_hbm, z_hbm, out_hbm,
                       ridx, cidx, bufs, acc, sem0, sem1, sem_i,
                       ssem0, ssem1):
        cid = lax.axis_index("c")
        sid = lax.axis_index("s")
        wid = sid * _NC + cid
        sems = (sem0, sem1)
        ssems = (ssem0, ssem1)

        base = pl.multiple_of(sid * rpt, 8)
        pltpu.sync_copy(z_hbm, acc.at[pl.ds(base, rpt)])
        plsc.subcore_barrier()

        lo = pl.multiple_of(wid * cpw, 8)
        cnt = lax.max(0, lax.min(nchunk_real - lo, cpw))

        def _istart(t, tb):
            off = pl.multiple_of(lo + t * _IB, 8)
            pltpu.async_copy(row_hbm.at[pl.ds(off, _IB)], ridx.at[tb], sem_i)
            pltpu.async_copy(col_hbm.at[pl.ds(off, _IB)], cidx.at[tb], sem_i)

        def _iwait(t, tb):
            off = pl.multiple_of(lo + t * _IB, 8)
            pltpu.make_async_copy(row_hbm.at[pl.ds(off, _IB)], ridx.at[tb],
                                  sem_i).wait()
            pltpu.make_async_copy(col_hbm.at[pl.ds(off, _IB)], cidx.at[tb],
                                  sem_i).wait()

        def _gstart(tb, i, b):
            pltpu.async_copy(y_hbm.at[ridx.at[tb, i]], bufs.at[b], sems[b])

        def _gwait(tb, i, b):
            pltpu.make_async_copy(y_hbm.at[ridx.at[tb, i]], bufs.at[b],
                                  sems[b]).wait()

        def _sstart(tb, i, b):
            pltpu.async_copy(bufs.at[b], acc.at[cidx.at[tb, i]], ssems[b],
                             add=True)

        def _swait(tb, i, b):
            pltpu.make_async_copy(bufs.at[b], acc.at[cidx.at[tb, i]],
                                  ssems[b]).wait()

        _istart(0, 0)
        for t in range(nstages):
            tb = t % 2
            _iwait(t, tb)
            if t + 1 < nstages:
                _istart(t + 1, 1 - tb)
            cnt_t = lax.max(0, lax.min(cnt - t * _IB, _IB))

            @pl.when(cnt_t > 0)
            def _():
                _gstart(tb, 0, 0)

            def _step(i2, _):
                for b in range(2):
                    i = i2 * 2 + b

                    @pl.when(i < cnt_t)
                    def _():
                        _gwait(tb, i, b)

                        @pl.when(i >= 1)
                        def _():
                            _swait(tb, i, 1 - b)

                        @pl.when(i + 1 < cnt_t)
                        def _():
                            _gstart(tb, i + 1, 1 - b)

                        _sstart(tb, i, b)
                return 0
            lax.fori_loop(0, _IB // 2, _step, 0)

            @pl.when((cnt_t > 0) & (cnt_t % 2 == 1))
            def _():
                _swait(tb, 0, 0)

            @pl.when((cnt_t > 0) & (cnt_t % 2 == 0))
            def _():
                _swait(tb, 0, 1)
        plsc.subcore_barrier()

        out_base = pl.multiple_of(cid * n_pad + base, 8)
        pltpu.sync_copy(acc.at[pl.ds(base, rpt)],
                        out_hbm.at[pl.ds(out_base, rpt)])

    return scatter_kernel(y, row_r, col_r, zeros)


def _deg_sc(row_r, ones, zeros, nchunk_real, n_pad, rpt):
    nchunk_pad = row_r.shape[0]
    d = ones.shape[1]
    cpw = nchunk_pad // _NW
    nstages = cpw // _IB
    assert nstages * _IB == cpw

    mesh = plsc.VectorSubcoreMesh(core_axis_name="c", subcore_axis_name="s")

    @functools.partial(
        pl.kernel,
        mesh=mesh,
        out_type=jax.ShapeDtypeStruct((2 * n_pad, d), jnp.float32),
        scratch_types=[
            pltpu.VMEM((2, _IB, _CHUNK), jnp.int32),
            pltpu.VMEM((_CHUNK, d), jnp.float32),
            pltpu.VMEM_SHARED((n_pad, d), jnp.float32),
            pltpu.SemaphoreType.DMA,
            pltpu.SemaphoreType.DMA,
            pltpu.SemaphoreType.DMA,
        ],
    )
    def deg_kernel(row_hbm, ones_hbm, z_hbm, out_hbm, ridx, onesb, acc,
                   sem_i, ssem0, ssem1):
        cid = lax.axis_index("c")
        sid = lax.axis_index("s")
        wid = sid * _NC + cid
        ssems = (ssem0, ssem1)

        base = pl.multiple_of(sid * rpt, 8)
        pltpu.sync_copy(z_hbm, acc.at[pl.ds(base, rpt)])
        pltpu.sync_copy(ones_hbm, onesb)
        plsc.subcore_barrier()

        lo = pl.multiple_of(wid * cpw, 8)
        cnt = lax.max(0, lax.min(nchunk_real - lo, cpw))

        def _istart(t, tb):
            off = pl.multiple_of(lo + t * _IB, 8)
            pltpu.async_copy(row_hbm.at[pl.ds(off, _IB)], ridx.at[tb], sem_i)

        def _iwait(t, tb):
            off = pl.multiple_of(lo + t * _IB, 8)
            pltpu.make_async_copy(row_hbm.at[pl.ds(off, _IB)], ridx.at[tb],
                                  sem_i).wait()

        def _sstart(tb, i, b):
            pltpu.async_copy(onesb, acc.at[ridx.at[tb, i]], ssems[b],
                             add=True)

        def _swait(tb, i, b):
            pltpu.make_async_copy(onesb, acc.at[ridx.at[tb, i]],
                                  ssems[b]).wait()

        _istart(0, 0)
        for t in range(nstages):
            tb = t % 2
            _iwait(t, tb)
            if t + 1 < nstages:
                _istart(t + 1, 1 - tb)
            cnt_t = lax.max(0, lax.min(cnt - t * _IB, _IB))

            def _step(i2, _):
                for b in range(2):
                    i = i2 * 2 + b

                    @pl.when(i < cnt_t)
                    def _():
                        @pl.when(i >= 2)
                        def _():
                            _swait(tb, i, b)

                        _sstart(tb, i, b)
                return 0
            lax.fori_loop(0, _IB // 2, _step, 0)

            @pl.when(cnt_t >= 2)
            def _():
                _swait(tb, 0, 0)
                _swait(tb, 0, 1)

            @pl.when(cnt_t == 1)
            def _():
                _swait(tb, 0, 0)
        plsc.subcore_barrier()

        out_base = pl.multiple_of(cid * n_pad + base, 8)
        pltpu.sync_copy(acc.at[pl.ds(base, rpt)],
                        out_hbm.at[pl.ds(out_base, rpt)])

    return deg_kernel(row_r, ones, zeros)


def _embed_proj_tc(x, w_t, lmat, deg):
    n, d = x.shape

    def body(x_ref, w_ref, l_ref, deg_ref, h_ref, dis_ref, y_ref):
        h = jnp.dot(x_ref[...], w_ref[...],
                    preferred_element_type=jnp.float32)
        h_ref[...] = h
        dg = deg_ref[...]
        dis = jnp.where(dg > 0, lax.rsqrt(dg), 0.0)
        dis_ref[...] = dis
        y_ref[...] = dis * jnp.dot(h, l_ref[...],
                                   preferred_element_type=jnp.float32)

    return pl.pallas_call(
        body,
        grid=(n // _BN,),
        in_specs=[
            pl.BlockSpec((_BN, d), lambda i: (i, 0)),
            pl.BlockSpec((d, d), lambda i: (0, 0)),
            pl.BlockSpec((d, d), lambda i: (0, 0)),
            pl.BlockSpec((_BN, 1), lambda i: (i, 0)),
        ],
        out_specs=[
            pl.BlockSpec((_BN, d), lambda i: (i, 0)),
            pl.BlockSpec((_BN, 1), lambda i: (i, 0)),
            pl.BlockSpec((_BN, d), lambda i: (i, 0)),
        ],
        out_shape=[
            jax.ShapeDtypeStruct((n, d), jnp.float32),
            jax.ShapeDtypeStruct((n, 1), jnp.float32),
            jax.ShapeDtypeStruct((n, d), jnp.float32),
        ],
    )(x, w_t, lmat, deg)


def _update_proj_tc(h, acc, dis, bias, wmat, lmat):
    n, d = h.shape

    def body(h_ref, acc_ref, dis_ref, b_ref, w_ref, l_ref, o_ref, y_ref):
        dis = dis_ref[...]
        agg = dis * (acc_ref[0] + acc_ref[1])
        hw = jnp.dot(h_ref[...], w_ref[...],
                     preferred_element_type=jnp.float32)
        hn = h_ref[...] + EPS * jnp.tanh(hw + agg + b_ref[...])
        o_ref[...] = hn
        y_ref[...] = dis * jnp.dot(hn, l_ref[...],
                                   preferred_element_type=jnp.float32)

    return pl.pallas_call(
        body,
        grid=(n // _BN,),
        in_specs=[
            pl.BlockSpec((_BN, d), lambda i: (i, 0)),
            pl.BlockSpec((2, _BN, d), lambda i: (0, i, 0)),
            pl.BlockSpec((_BN, 1), lambda i: (i, 0)),
            pl.BlockSpec((1, d), lambda i: (0, 0)),
            pl.BlockSpec((d, d), lambda i: (0, 0)),
            pl.BlockSpec((d, d), lambda i: (0, 0)),
        ],
        out_specs=[
            pl.BlockSpec((_BN, d), lambda i: (i, 0)),
            pl.BlockSpec((_BN, d), lambda i: (i, 0)),
        ],
        out_shape=[
            jax.ShapeDtypeStruct((n, d), jnp.float32),
            jax.ShapeDtypeStruct((n, d), jnp.float32),
        ],
    )(h, acc, dis, bias, wmat, lmat)


def _update_tc(h, acc, dis, bias, wmat):
    n, d = h.shape

    def body(h_ref, acc_ref, dis_ref, b_ref, w_ref, o_ref):
        agg = dis_ref[...] * (acc_ref[0] + acc_ref[1])
        hw = jnp.dot(h_ref[...], w_ref[...],
                     preferred_element_type=jnp.float32)
        o_ref[...] = h_ref[...] + EPS * jnp.tanh(hw + agg + b_ref[...])

    return pl.pallas_call(
        body,
        grid=(n // _BN,),
        in_specs=[
            pl.BlockSpec((_BN, d), lambda i: (i, 0)),
            pl.BlockSpec((2, _BN, d), lambda i: (0, i, 0)),
            pl.BlockSpec((_BN, 1), lambda i: (i, 0)),
            pl.BlockSpec((1, d), lambda i: (0, 0)),
            pl.BlockSpec((d, d), lambda i: (0, 0)),
        ],
        out_specs=pl.BlockSpec((_BN, d), lambda i: (i, 0)),
        out_shape=jax.ShapeDtypeStruct((n, d), jnp.float32),
    )(h, acc, dis, bias, wmat)


def kernel(x, edge_index, emb_w, Weights, biases, lin_ws):
    n, d = x.shape
    e = edge_index.shape[1]
    nlayers = Weights.shape[0]

    nchunk = e // _CHUNK
    cpw = _pad_chunks(nchunk)
    nchunk_pad = cpw * _NW
    n_pad, rpt = _pad_nodes(n)

    row_r = edge_index[0].reshape(nchunk, _CHUNK)
    col_r = edge_index[1].reshape(nchunk, _CHUNK)
    pad = ((0, nchunk_pad - nchunk), (0, 0))
    row_r = jnp.pad(row_r, pad)
    col_r = jnp.pad(col_r, pad)

    eye = jnp.eye(d, dtype=jnp.float32)
    wmats = jnp.transpose(Weights, (0, 2, 1)) - Weights - GAMMA * eye
    lmats = jnp.transpose(lin_ws, (0, 2, 1))

    zeros = jnp.zeros((rpt, d), jnp.float32)
    ones = jnp.ones((_CHUNK, d), jnp.float32)
    degp = _deg_sc(row_r, ones, zeros, nchunk, n_pad, rpt)
    deg = degp[:n, :1] + degp[n_pad:n_pad + n, :1]

    h, dis, y = _embed_proj_tc(x, emb_w.T, lmats[0], deg)
    for l in range(nlayers):
        accp = _scatter_add_sc(y, row_r, col_r, zeros, nchunk, n_pad, rpt)
        acc = jnp.stack([accp[:n], accp[n_pad:n_pad + n]])
        if l + 1 < nlayers:
            h, y = _update_proj_tc(h, acc, dis, biases[l].reshape(1, d),
                                   wmats[l], lmats[l + 1])
        else:
            h = _update_tc(h, acc, dis, biases[l].reshape(1, d), wmats[l])
    return h


# speedup vs baseline: 1.1059x; 1.0436x over previous
import functools

import jax
import jax.numpy as jnp
from jax import lax
from jax.experimental import pallas as pl
from jax.experimental.pallas import tpu as pltpu
from jax.experimental.pallas import tpu_sc as plsc

GAMMA = 0.1
EPS = 0.1

_CHUNK = 128
_NC = 2
_NS = 16
_NW = _NC * _NS
_BN = 1000


def _pad_chunks(nchunk):
    cpw = -(-nchunk // _NW)
    return -(-cpw // 8) * 8


def _pad_nodes(n_nodes):
    per_tile = -(-n_nodes // _NS)
    per_tile = -(-per_tile // 128) * 128
    return per_tile * _NS, per_tile


_IB = 16


def _scatter_add_sc(y, row_r, col_r, zeros, nchunk_real, n_pad, rpt):
    nchunk_pad = row_r.shape[0]
    d = y.shape[1]
    cpw = nchunk_pad // _NW
    nstages = cpw // _IB
    assert nstages * _IB == cpw

    mesh = plsc.VectorSubcoreMesh(core_axis_name="c", subcore_axis_name="s")

    @functools.partial(
        pl.kernel,
        mesh=mesh,
        out_type=jax.ShapeDtypeStruct((2 * n_pad, d), jnp.float32),
        scratch_types=[
            pltpu.VMEM((2, _IB, _CHUNK), jnp.int32),
            pltpu.VMEM((2, _IB, _CHUNK), jnp.int32),
            pltpu.VMEM((2, _CHUNK, d), jnp.float32),
            pltpu.VMEM_SHARED((n_pad, d), jnp.float32),
            pltpu.SemaphoreType.DMA,
            pltpu.SemaphoreType.DMA,
            pltpu.SemaphoreType.DMA,
            pltpu.SemaphoreType.DMA,
            pltpu.SemaphoreType.DMA,
        ],
    )
    def scatter_kernel(y_hbm, row_hbm, col_hbm, z_hbm, out_hbm,
                       ridx, cidx, bufs, acc, sem0, sem1, sem_i,
                       ssem0, ssem1):
        cid = lax.axis_index("c")
        sid = lax.axis_index("s")
        wid = sid * _NC + cid
        sems = (sem0, sem1)
        ssems = (ssem0, ssem1)

        base = pl.multiple_of(sid * rpt, 8)
        pltpu.sync_copy(z_hbm, acc.at[pl.ds(base, rpt)])
        plsc.subcore_barrier()

        lo = pl.multiple_of(wid * cpw, 8)
        cnt = lax.max(0, lax.min(nchunk_real - lo, cpw))

        def _istart(t, tb):
            off = pl.multiple_of(lo + t * _IB, 8)
            pltpu.async_copy(row_hbm.at[pl.ds(off, _IB)], ridx.at[tb], sem_i)
            pltpu.async_copy(col_hbm.at[pl.ds(off, _IB)], cidx.at[tb], sem_i)

        def _iwait(t, tb):
            off = pl.multiple_of(lo + t * _IB, 8)
            pltpu.make_async_copy(row_hbm.at[pl.ds(off, _IB)], ridx.at[tb],
                                  sem_i).wait()
            pltpu.make_async_copy(col_hbm.at[pl.ds(off, _IB)], cidx.at[tb],
                                  sem_i).wait()

        def _gstart(tb, i, b):
            pltpu.async_copy(y_hbm.at[ridx.at[tb, i]], bufs.at[b], sems[b])

        def _gwait(tb, i, b):
            pltpu.make_async_copy(y_hbm.at[ridx.at[tb, i]], bufs.at[b],
                                  sems[b]).wait()

        def _sstart(tb, i, b):
            pltpu.async_copy(bufs.at[b], acc.at[cidx.at[tb, i]], ssems[b],
                             add=True)

        def _swait(tb, i, b):
            pltpu.make_async_copy(bufs.at[b], acc.at[cidx.at[tb, i]],
                                  ssems[b]).wait()

        _istart(0, 0)
        for t in range(nstages):
            tb = t % 2
            _iwait(t, tb)
            if t + 1 < nstages:
                _istart(t + 1, 1 - tb)
            cnt_t = lax.max(0, lax.min(cnt - t * _IB, _IB))

            @pl.when(cnt_t > 0)
            def _():
                _gstart(tb, 0, 0)

            def _step(i2, _):
                for b in range(2):
                    i = i2 * 2 + b

                    @pl.when(i < cnt_t)
                    def _():
                        _gwait(tb, i, b)

                        @pl.when(i >= 1)
                        def _():
                            _swait(tb, i, 1 - b)

                        @pl.when(i + 1 < cnt_t)
                        def _():
                            _gstart(tb, i + 1, 1 - b)

                        _sstart(tb, i, b)
                return 0
            lax.fori_loop(0, _IB // 2, _step, 0)

            @pl.when((cnt_t > 0) & (cnt_t % 2 == 1))
            def _():
                _swait(tb, 0, 0)

            @pl.when((cnt_t > 0) & (cnt_t % 2 == 0))
            def _():
                _swait(tb, 0, 1)
        plsc.subcore_barrier()

        out_base = pl.multiple_of(cid * n_pad + base, 8)
        pltpu.sync_copy(acc.at[pl.ds(base, rpt)],
                        out_hbm.at[pl.ds(out_base, rpt)])

    return scatter_kernel(y, row_r, col_r, zeros)


def _deg_sc(row_r, ones, zeros, nchunk_real, n_pad, rpt):
    nchunk_pad = row_r.shape[0]
    d = ones.shape[1]
    cpw = nchunk_pad // _NW
    nstages = cpw // _IB
    assert nstages * _IB == cpw

    mesh = plsc.VectorSubcoreMesh(core_axis_name="c", subcore_axis_name="s")

    @functools.partial(
        pl.kernel,
        mesh=mesh,
        out_type=jax.ShapeDtypeStruct((2 * n_pad, d), jnp.float32),
        scratch_types=[
            pltpu.VMEM((2, _IB, _CHUNK), jnp.int32),
            pltpu.VMEM((_CHUNK, d), jnp.float32),
            pltpu.VMEM_SHARED((n_pad, d), jnp.float32),
            pltpu.SemaphoreType.DMA,
            pltpu.SemaphoreType.DMA,
            pltpu.SemaphoreType.DMA,
        ],
    )
    def deg_kernel(row_hbm, ones_hbm, z_hbm, out_hbm, ridx, onesb, acc,
                   sem_i, ssem0, ssem1):
        cid = lax.axis_index("c")
        sid = lax.axis_index("s")
        wid = sid * _NC + cid
        ssems = (ssem0, ssem1)

        base = pl.multiple_of(sid * rpt, 8)
        pltpu.sync_copy(z_hbm, acc.at[pl.ds(base, rpt)])
        pltpu.sync_copy(ones_hbm, onesb)
        plsc.subcore_barrier()

        lo = pl.multiple_of(wid * cpw, 8)
        cnt = lax.max(0, lax.min(nchunk_real - lo, cpw))

        def _istart(t, tb):
            off = pl.multiple_of(lo + t * _IB, 8)
            pltpu.async_copy(row_hbm.at[pl.ds(off, _IB)], ridx.at[tb], sem_i)

        def _iwait(t, tb):
            off = pl.multiple_of(lo + t * _IB, 8)
            pltpu.make_async_copy(row_hbm.at[pl.ds(off, _IB)], ridx.at[tb],
                                  sem_i).wait()

        def _sstart(tb, i, b):
            pltpu.async_copy(onesb, acc.at[ridx.at[tb, i]], ssems[b],
                             add=True)

        def _swait(tb, i, b):
            pltpu.make_async_copy(onesb, acc.at[ridx.at[tb, i]],
                                  ssems[b]).wait()

        _istart(0, 0)
        for t in range(nstages):
            tb = t % 2
            _iwait(t, tb)
            if t + 1 < nstages:
                _istart(t + 1, 1 - tb)
            cnt_t = lax.max(0, lax.min(cnt - t * _IB, _IB))

            def _step(i2, _):
                for b in range(2):
                    i = i2 * 2 + b

                    @pl.when(i < cnt_t)
                    def _():
                        @pl.when(i >= 2)
                        def _():
                            _swait(tb, i, b)

                        _sstart(tb, i, b)
                return 0
            lax.fori_loop(0, _IB // 2, _step, 0)

            @pl.when(cnt_t >= 2)
            def _():
                _swait(tb, 0, 0)
                _swait(tb, 0, 1)

            @pl.when(cnt_t == 1)
            def _():
                _swait(tb, 0, 0)
        plsc.subcore_barrier()

        out_base = pl.multiple_of(cid * n_pad + base, 8)
        pltpu.sync_copy(acc.at[pl.ds(base, rpt)],
                        out_hbm.at[pl.ds(out_base, rpt)])

    return deg_kernel(row_r, ones, zeros)


def _embed_proj_tc(x, w_t, lmat, deg):
    n, d = x.shape

    def body(x_ref, w_ref, l_ref, deg_ref, h_ref, dis_ref, y_ref):
        h = jnp.dot(x_ref[...], w_ref[...],
                    preferred_element_type=jnp.float32)
        h_ref[...] = h
        dg = deg_ref[...]
        dis = jnp.where(dg > 0, lax.rsqrt(dg), 0.0)
        dis_ref[...] = dis
        y_ref[...] = dis * jnp.dot(h, l_ref[...],
                                   preferred_element_type=jnp.float32)

    return pl.pallas_call(
        body,
        grid=(n // _BN,),
        in_specs=[
            pl.BlockSpec((_BN, d), lambda i: (i, 0)),
            pl.BlockSpec((d, d), lambda i: (0, 0)),
            pl.BlockSpec((d, d), lambda i: (0, 0)),
            pl.BlockSpec((_BN, 1), lambda i: (i, 0)),
        ],
        out_specs=[
            pl.BlockSpec((_BN, d), lambda i: (i, 0)),
            pl.BlockSpec((_BN, 1), lambda i: (i, 0)),
            pl.BlockSpec((_BN, d), lambda i: (i, 0)),
        ],
        out_shape=[
            jax.ShapeDtypeStruct((n, d), jnp.float32),
            jax.ShapeDtypeStruct((n, 1), jnp.float32),
            jax.ShapeDtypeStruct((n, d), jnp.float32),
        ],
    )(x, w_t, lmat, deg)


def _update_proj_tc(h, acc, dis, bias, wmat, lmat):
    n, d = h.shape

    def body(h_ref, a0_ref, a1_ref, dis_ref, b_ref, w_ref, l_ref,
             o_ref, y_ref):
        dis = dis_ref[...]
        agg = dis * (a0_ref[0] + a1_ref[0])
        hw = jnp.dot(h_ref[...], w_ref[...],
                     preferred_element_type=jnp.float32)
        hn = h_ref[...] + EPS * jnp.tanh(hw + agg + b_ref[...])
        o_ref[...] = hn
        y_ref[...] = dis * jnp.dot(hn, l_ref[...],
                                   preferred_element_type=jnp.float32)

    return pl.pallas_call(
        body,
        grid=(n // _BN,),
        in_specs=[
            pl.BlockSpec((_BN, d), lambda i: (i, 0)),
            pl.BlockSpec((1, _BN, d), lambda i: (0, i, 0)),
            pl.BlockSpec((1, _BN, d), lambda i: (1, i, 0)),
            pl.BlockSpec((_BN, 1), lambda i: (i, 0)),
            pl.BlockSpec((1, d), lambda i: (0, 0)),
            pl.BlockSpec((d, d), lambda i: (0, 0)),
            pl.BlockSpec((d, d), lambda i: (0, 0)),
        ],
        out_specs=[
            pl.BlockSpec((_BN, d), lambda i: (i, 0)),
            pl.BlockSpec((_BN, d), lambda i: (i, 0)),
        ],
        out_shape=[
            jax.ShapeDtypeStruct((n, d), jnp.float32),
            jax.ShapeDtypeStruct((n, d), jnp.float32),
        ],
    )(h, acc, acc, dis, bias, wmat, lmat)


def _update_tc(h, acc, dis, bias, wmat):
    n, d = h.shape

    def body(h_ref, a0_ref, a1_ref, dis_ref, b_ref, w_ref, o_ref):
        agg = dis_ref[...] * (a0_ref[0] + a1_ref[0])
        hw = jnp.dot(h_ref[...], w_ref[...],
                     preferred_element_type=jnp.float32)
        o_ref[...] = h_ref[...] + EPS * jnp.tanh(hw + agg + b_ref[...])

    return pl.pallas_call(
        body,
        grid=(n // _BN,),
        in_specs=[
            pl.BlockSpec((_BN, d), lambda i: (i, 0)),
            pl.BlockSpec((1, _BN, d), lambda i: (0, i, 0)),
            pl.BlockSpec((1, _BN, d), lambda i: (1, i, 0)),
            pl.BlockSpec((_BN, 1), lambda i: (i, 0)),
            pl.BlockSpec((1, d), lambda i: (0, 0)),
            pl.BlockSpec((d, d), lambda i: (0, 0)),
        ],
        out_specs=pl.BlockSpec((_BN, d), lambda i: (i, 0)),
        out_shape=jax.ShapeDtypeStruct((n, d), jnp.float32),
    )(h, acc, acc, dis, bias, wmat)


def kernel(x, edge_index, emb_w, Weights, biases, lin_ws):
    n, d = x.shape
    e = edge_index.shape[1]
    nlayers = Weights.shape[0]

    nchunk = e // _CHUNK
    cpw = _pad_chunks(nchunk)
    nchunk_pad = cpw * _NW
    n_pad, rpt = _pad_nodes(n)

    row_r = edge_index[0].reshape(nchunk, _CHUNK)
    col_r = edge_index[1].reshape(nchunk, _CHUNK)
    pad = ((0, nchunk_pad - nchunk), (0, 0))
    row_r = jnp.pad(row_r, pad)
    col_r = jnp.pad(col_r, pad)

    eye = jnp.eye(d, dtype=jnp.float32)
    wmats = jnp.transpose(Weights, (0, 2, 1)) - Weights - GAMMA * eye
    lmats = jnp.transpose(lin_ws, (0, 2, 1))

    zeros = jnp.zeros((rpt, d), jnp.float32)
    ones = jnp.ones((_CHUNK, d), jnp.float32)
    degp = _deg_sc(row_r, ones, zeros, nchunk, n_pad, rpt)
    deg = degp[:n, :1] + degp[n_pad:n_pad + n, :1]

    h, dis, y = _embed_proj_tc(x, emb_w.T, lmats[0], deg)
    for l in range(nlayers):
        accp = _scatter_add_sc(y, row_r, col_r, zeros, nchunk, n_pad, rpt)
        acc = accp.reshape(2, n_pad, d)
        if l + 1 < nlayers:
            h, y = _update_proj_tc(h, acc, dis, biases[l].reshape(1, d),
                                   wmats[l], lmats[l + 1])
        else:
            h = _update_tc(h, acc, dis, biases[l].reshape(1, d), wmats[l])
    return h


# speedup vs baseline: 1.2096x; 1.0938x over previous
import functools

import jax
import jax.numpy as jnp
from jax import lax
from jax.experimental import pallas as pl
from jax.experimental.pallas import tpu as pltpu
from jax.experimental.pallas import tpu_sc as plsc

GAMMA = 0.1
EPS = 0.1

_CHUNK = 128
_NC = 2
_NS = 16
_NW = _NC * _NS
_BN = 1000


def _pad_chunks(nchunk):
    cpw = -(-nchunk // _NW)
    return -(-cpw // 8) * 8


def _pad_nodes(n_nodes):
    per_tile = -(-n_nodes // _NS)
    per_tile = -(-per_tile // 128) * 128
    return per_tile * _NS, per_tile


_IB = 16


def _scatter_add_sc(y, row_r, col_r, zeros, nchunk_real, n_pad, rpt):
    nchunk_pad = row_r.shape[0]
    d = y.shape[1]
    cpw = nchunk_pad // _NW
    nstages = cpw // _IB
    assert nstages * _IB == cpw

    mesh = plsc.VectorSubcoreMesh(core_axis_name="c", subcore_axis_name="s")

    @functools.partial(
        pl.kernel,
        mesh=mesh,
        out_type=jax.ShapeDtypeStruct((2 * n_pad, d), jnp.float32),
        scratch_types=[
            pltpu.VMEM((2, _IB, _CHUNK), jnp.int32),
            pltpu.VMEM((2, _IB, _CHUNK), jnp.int32),
            pltpu.VMEM((2, _CHUNK, d), jnp.float32),
            pltpu.VMEM_SHARED((n_pad, d), jnp.float32),
            pltpu.SemaphoreType.DMA,
            pltpu.SemaphoreType.DMA,
            pltpu.SemaphoreType.DMA,
            pltpu.SemaphoreType.DMA,
            pltpu.SemaphoreType.DMA,
        ],
    )
    def scatter_kernel(y_hbm, row_hbm, col_hbm, z_hbm, out_hbm,
                       ridx, cidx, bufs, acc, sem0, sem1, sem_i,
                       ssem0, ssem1):
        cid = lax.axis_index("c")
        sid = lax.axis_index("s")
        wid = sid * _NC + cid
        sems = (sem0, sem1)
        ssems = (ssem0, ssem1)

        base = pl.multiple_of(sid * rpt, 8)
        pltpu.sync_copy(z_hbm, acc.at[pl.ds(base, rpt)])
        plsc.subcore_barrier()

        lo = pl.multiple_of(wid * cpw, 8)
        cnt = lax.max(0, lax.min(nchunk_real - lo, cpw))

        def _istart(t, tb):
            off = pl.multiple_of(lo + t * _IB, 8)
            pltpu.async_copy(row_hbm.at[pl.ds(off, _IB)], ridx.at[tb], sem_i)
            pltpu.async_copy(col_hbm.at[pl.ds(off, _IB)], cidx.at[tb], sem_i)

        def _iwait(t, tb):
            off = pl.multiple_of(lo + t * _IB, 8)
            pltpu.make_async_copy(row_hbm.at[pl.ds(off, _IB)], ridx.at[tb],
                                  sem_i).wait()
            pltpu.make_async_copy(col_hbm.at[pl.ds(off, _IB)], cidx.at[tb],
                                  sem_i).wait()

        def _gstart(tb, i, b):
            pltpu.async_copy(y_hbm.at[ridx.at[tb, i]], bufs.at[b], sems[b])

        def _gwait(tb, i, b):
            pltpu.make_async_copy(y_hbm.at[ridx.at[tb, i]], bufs.at[b],
                                  sems[b]).wait()

        def _sstart(tb, i, b):
            pltpu.async_copy(bufs.at[b], acc.at[cidx.at[tb, i]], ssems[b],
                             add=True)

        def _swait(tb, i, b):
            pltpu.make_async_copy(bufs.at[b], acc.at[cidx.at[tb, i]],
                                  ssems[b]).wait()

        _istart(0, 0)
        for t in range(nstages):
            tb = t % 2
            _iwait(t, tb)
            if t + 1 < nstages:
                _istart(t + 1, 1 - tb)
            cnt_t = lax.max(0, lax.min(cnt - t * _IB, _IB))

            @pl.when(cnt_t > 0)
            def _():
                _gstart(tb, 0, 0)

            def _step(i2, _):
                for b in range(2):
                    i = i2 * 2 + b

                    @pl.when(i < cnt_t)
                    def _():
                        _gwait(tb, i, b)

                        @pl.when(i >= 1)
                        def _():
                            _swait(tb, i, 1 - b)

                        @pl.when(i + 1 < cnt_t)
                        def _():
                            _gstart(tb, i + 1, 1 - b)

                        _sstart(tb, i, b)
                return 0
            lax.fori_loop(0, _IB // 2, _step, 0)

            @pl.when((cnt_t > 0) & (cnt_t % 2 == 1))
            def _():
                _swait(tb, 0, 0)

            @pl.when((cnt_t > 0) & (cnt_t % 2 == 0))
            def _():
                _swait(tb, 0, 1)
        plsc.subcore_barrier()

        out_base = pl.multiple_of(cid * n_pad + base, 8)
        pltpu.sync_copy(acc.at[pl.ds(base, rpt)],
                        out_hbm.at[pl.ds(out_base, rpt)])

    return scatter_kernel(y, row_r, col_r, zeros)


def _deg_sc(row_r, ones, zeros, nchunk_real, n_pad, rpt):
    nchunk_pad = row_r.shape[0]
    cpw = nchunk_pad // _NW
    nstages = cpw // _IB
    assert nstages * _IB == cpw

    mesh = plsc.VectorSubcoreMesh(core_axis_name="c", subcore_axis_name="s")

    @functools.partial(
        pl.kernel,
        mesh=mesh,
        out_type=jax.ShapeDtypeStruct((2 * n_pad,), jnp.float32),
        scratch_types=[
            pltpu.VMEM((2, _IB, _CHUNK), jnp.int32),
            pltpu.VMEM((_CHUNK,), jnp.float32),
            pltpu.VMEM_SHARED((n_pad,), jnp.float32),
            pltpu.SemaphoreType.DMA,
            pltpu.SemaphoreType.DMA,
            pltpu.SemaphoreType.DMA,
        ],
    )
    def deg_kernel(row_hbm, ones_hbm, z_hbm, out_hbm, ridx, onesb, acc,
                   sem_i, ssem0, ssem1):
        cid = lax.axis_index("c")
        sid = lax.axis_index("s")
        wid = sid * _NC + cid
        ssems = (ssem0, ssem1)

        base = pl.multiple_of(sid * rpt, 8)
        pltpu.sync_copy(z_hbm, acc.at[pl.ds(base, rpt)])
        pltpu.sync_copy(ones_hbm, onesb)
        plsc.subcore_barrier()

        lo = pl.multiple_of(wid * cpw, 8)
        cnt = lax.max(0, lax.min(nchunk_real - lo, cpw))

        def _istart(t, tb):
            off = pl.multiple_of(lo + t * _IB, 8)
            pltpu.async_copy(row_hbm.at[pl.ds(off, _IB)], ridx.at[tb], sem_i)

        def _iwait(t, tb):
            off = pl.multiple_of(lo + t * _IB, 8)
            pltpu.make_async_copy(row_hbm.at[pl.ds(off, _IB)], ridx.at[tb],
                                  sem_i).wait()

        def _sstart(tb, i, b):
            pltpu.async_copy(onesb, acc.at[ridx.at[tb, i]], ssems[b],
                             add=True)

        def _swait(tb, i, b):
            pltpu.make_async_copy(onesb, acc.at[ridx.at[tb, i]],
                                  ssems[b]).wait()

        _istart(0, 0)
        for t in range(nstages):
            tb = t % 2
            _iwait(t, tb)
            if t + 1 < nstages:
                _istart(t + 1, 1 - tb)
            cnt_t = lax.max(0, lax.min(cnt - t * _IB, _IB))

            def _step(i2, _):
                for b in range(2):
                    i = i2 * 2 + b

                    @pl.when(i < cnt_t)
                    def _():
                        @pl.when(i >= 2)
                        def _():
                            _swait(tb, i, b)

                        _sstart(tb, i, b)
                return 0
            lax.fori_loop(0, _IB // 2, _step, 0)

            @pl.when(cnt_t >= 2)
            def _():
                _swait(tb, 0, 0)
                _swait(tb, 0, 1)

            @pl.when(cnt_t == 1)
            def _():
                _swait(tb, 0, 0)
        plsc.subcore_barrier()

        out_base = pl.multiple_of(cid * n_pad + base, 8)
        pltpu.sync_copy(acc.at[pl.ds(base, rpt)],
                        out_hbm.at[pl.ds(out_base, rpt)])

    return deg_kernel(row_r, ones, zeros)


def _embed_proj_tc(x, w_t, lmat, deg):
    n, d = x.shape

    def body(x_ref, w_ref, l_ref, deg_ref, h_ref, dis_ref, y_ref):
        h = jnp.dot(x_ref[...], w_ref[...],
                    preferred_element_type=jnp.float32)
        h_ref[...] = h
        dg = deg_ref[...]
        dis = jnp.where(dg > 0, lax.rsqrt(dg), 0.0)
        dis_ref[...] = dis
        y_ref[...] = dis * jnp.dot(h, l_ref[...],
                                   preferred_element_type=jnp.float32)

    return pl.pallas_call(
        body,
        grid=(n // _BN,),
        in_specs=[
            pl.BlockSpec((_BN, d), lambda i: (i, 0)),
            pl.BlockSpec((d, d), lambda i: (0, 0)),
            pl.BlockSpec((d, d), lambda i: (0, 0)),
            pl.BlockSpec((_BN, 1), lambda i: (i, 0)),
        ],
        out_specs=[
            pl.BlockSpec((_BN, d), lambda i: (i, 0)),
            pl.BlockSpec((_BN, 1), lambda i: (i, 0)),
            pl.BlockSpec((_BN, d), lambda i: (i, 0)),
        ],
        out_shape=[
            jax.ShapeDtypeStruct((n, d), jnp.float32),
            jax.ShapeDtypeStruct((n, 1), jnp.float32),
            jax.ShapeDtypeStruct((n, d), jnp.float32),
        ],
    )(x, w_t, lmat, deg)


def _update_proj_tc(h, acc, dis, bias, wmat, lmat):
    n, d = h.shape

    def body(h_ref, a0_ref, a1_ref, dis_ref, b_ref, w_ref, l_ref,
             o_ref, y_ref):
        dis = dis_ref[...]
        agg = dis * (a0_ref[0] + a1_ref[0])
        hw = jnp.dot(h_ref[...], w_ref[...],
                     preferred_element_type=jnp.float32)
        hn = h_ref[...] + EPS * jnp.tanh(hw + agg + b_ref[...])
        o_ref[...] = hn
        y_ref[...] = dis * jnp.dot(hn, l_ref[...],
                                   preferred_element_type=jnp.float32)

    return pl.pallas_call(
        body,
        grid=(n // _BN,),
        in_specs=[
            pl.BlockSpec((_BN, d), lambda i: (i, 0)),
            pl.BlockSpec((1, _BN, d), lambda i: (0, i, 0)),
            pl.BlockSpec((1, _BN, d), lambda i: (1, i, 0)),
            pl.BlockSpec((_BN, 1), lambda i: (i, 0)),
            pl.BlockSpec((1, d), lambda i: (0, 0)),
            pl.BlockSpec((d, d), lambda i: (0, 0)),
            pl.BlockSpec((d, d), lambda i: (0, 0)),
        ],
        out_specs=[
            pl.BlockSpec((_BN, d), lambda i: (i, 0)),
            pl.BlockSpec((_BN, d), lambda i: (i, 0)),
        ],
        out_shape=[
            jax.ShapeDtypeStruct((n, d), jnp.float32),
            jax.ShapeDtypeStruct((n, d), jnp.float32),
        ],
    )(h, acc, acc, dis, bias, wmat, lmat)


def _update_tc(h, acc, dis, bias, wmat):
    n, d = h.shape

    def body(h_ref, a0_ref, a1_ref, dis_ref, b_ref, w_ref, o_ref):
        agg = dis_ref[...] * (a0_ref[0] + a1_ref[0])
        hw = jnp.dot(h_ref[...], w_ref[...],
                     preferred_element_type=jnp.float32)
        o_ref[...] = h_ref[...] + EPS * jnp.tanh(hw + agg + b_ref[...])

    return pl.pallas_call(
        body,
        grid=(n // _BN,),
        in_specs=[
            pl.BlockSpec((_BN, d), lambda i: (i, 0)),
            pl.BlockSpec((1, _BN, d), lambda i: (0, i, 0)),
            pl.BlockSpec((1, _BN, d), lambda i: (1, i, 0)),
            pl.BlockSpec((_BN, 1), lambda i: (i, 0)),
            pl.BlockSpec((1, d), lambda i: (0, 0)),
            pl.BlockSpec((d, d), lambda i: (0, 0)),
        ],
        out_specs=pl.BlockSpec((_BN, d), lambda i: (i, 0)),
        out_shape=jax.ShapeDtypeStruct((n, d), jnp.float32),
    )(h, acc, acc, dis, bias, wmat)


def kernel(x, edge_index, emb_w, Weights, biases, lin_ws):
    n, d = x.shape
    e = edge_index.shape[1]
    nlayers = Weights.shape[0]

    nchunk = e // _CHUNK
    cpw = _pad_chunks(nchunk)
    nchunk_pad = cpw * _NW
    n_pad, rpt = _pad_nodes(n)

    row_r = edge_index[0].reshape(nchunk, _CHUNK)
    col_r = edge_index[1].reshape(nchunk, _CHUNK)
    pad = ((0, nchunk_pad - nchunk), (0, 0))
    row_r = jnp.pad(row_r, pad)
    col_r = jnp.pad(col_r, pad)

    eye = jnp.eye(d, dtype=jnp.float32)
    wmats = jnp.transpose(Weights, (0, 2, 1)) - Weights - GAMMA * eye
    lmats = jnp.transpose(lin_ws, (0, 2, 1))

    zeros = jnp.zeros((rpt, d), jnp.float32)
    ones1 = jnp.ones((_CHUNK,), jnp.float32)
    zeros1 = jnp.zeros((rpt,), jnp.float32)
    degp = _deg_sc(row_r, ones1, zeros1, nchunk, n_pad, rpt)
    deg = (degp[:n] + degp[n_pad:n_pad + n]).reshape(n, 1)

    h, dis, y = _embed_proj_tc(x, emb_w.T, lmats[0], deg)
    for l in range(nlayers):
        accp = _scatter_add_sc(y, row_r, col_r, zeros, nchunk, n_pad, rpt)
        acc = accp.reshape(2, n_pad, d)
        if l + 1 < nlayers:
            h, y = _update_proj_tc(h, acc, dis, biases[l].reshape(1, d),
                                   wmats[l], lmats[l + 1])
        else:
            h = _update_tc(h, acc, dis, biases[l].reshape(1, d), wmats[l])
    return h
